# baseline (speedup 1.0000x reference)
import functools

import jax
import jax.numpy as jnp
from jax import lax
from jax.experimental import pallas as pl
from jax.experimental.pallas import tpu as pltpu
from jax.experimental.pallas import tpu_sc as plsc

D = 64
NC, NS = 2, 16
NW = NC * NS
CHUNK = 128


def _gather_body(n_chunk, x_hbm, table_hbm, out_hbm, idx_v, rows_v, sem_i, sem_g):
    wid = lax.axis_index("s") * NC + lax.axis_index("c")
    b_per_w = n_chunk * CHUNK
    base = wid * b_per_w
    pltpu.async_copy(x_hbm.at[wid], idx_v, sem_i).wait()

    def body(j, _):
        pltpu.async_copy(table_hbm.at[idx_v.at[j]], rows_v, sem_g).wait()
        pltpu.sync_copy(rows_v, out_hbm.at[pl.ds(base + j * CHUNK, CHUNK)])
        return 0

    lax.fori_loop(0, n_chunk, body, 0)


def _build(n_total, n_chunk):
    mesh = plsc.VectorSubcoreMesh(core_axis_name="c", subcore_axis_name="s")
    return pl.kernel(
        functools.partial(_gather_body, n_chunk),
        out_type=jax.ShapeDtypeStruct((n_total, D), jnp.float32),
        mesh=mesh,
        scratch_types=[
            pltpu.VMEM((n_chunk, CHUNK), jnp.int32),
            pltpu.VMEM((CHUNK, D), jnp.float32),
            pltpu.SemaphoreType.DMA,
            pltpu.SemaphoreType.DMA,
        ],
        compiler_params=pltpu.CompilerParams(use_tc_tiling_on_sc=False),
    )


def kernel(x, weight):
    orig_shape = x.shape
    n_total = x.size
    b_per_w = n_total // NW
    n_chunk = b_per_w // CHUNK
    xw = x.reshape(NW, n_chunk, CHUNK).astype(jnp.int32)
    out = _build(n_total, n_chunk)(xw, weight)
    return out.astype(jnp.bfloat16).reshape(*orig_shape, D)


# speedup vs baseline: 1.0582x; 1.0582x over previous
import functools

import jax
import jax.numpy as jnp
from jax import lax
from jax.experimental import pallas as pl
from jax.experimental.pallas import tpu as pltpu
from jax.experimental.pallas import tpu_sc as plsc

D = 64
NC, NS = 2, 16
NW = NC * NS
CHUNK = 128
L = 16


def _cast_chunk(rows_ref, out_ref):
    iota = lax.iota(jnp.int32, L)
    even = 2 * iota
    odd = even + 1

    def row_body(r, _):
        rv = jnp.full((L,), r, jnp.int32)
        for h in range(D // (2 * L)):
            c0 = h * 2 * L
            e = plsc.load_gather(rows_ref, [rv, c0 + even])
            o = plsc.load_gather(rows_ref, [rv, c0 + odd])
            pk = plsc.pack(e, o, format=plsc.PackFormat.INTERLEAVED)
            out_ref[r, pl.ds(c0, 2 * L)] = pk
        return 0

    lax.fori_loop(0, CHUNK, row_body, 0, unroll=4)


def _emb_body(n_chunk, x_hbm, w_hbm, out_hbm, idx_v, r0, r1, o0, o1,
              sem_i, g0, g1, w0, w1):
    wid = lax.axis_index("s") * NC + lax.axis_index("c")
    base = wid * n_chunk * CHUNK
    pltpu.async_copy(x_hbm.at[wid], idx_v, sem_i).wait()

    rows = (r0, r1)
    outs = (o0, o1)
    gs = (g0, g1)
    ws = (w0, w1)

    def fire_gather(c, b):
        pltpu.async_copy(w_hbm.at[idx_v.at[c]], rows[b], gs[b])

    def wait_gather(c, b):
        pltpu.make_async_copy(w_hbm.at[idx_v.at[c]], rows[b], gs[b]).wait()

    def fire_write(c, b):
        pltpu.async_copy(outs[b], out_hbm.at[pl.ds(base + c * CHUNK, CHUNK)],
                         ws[b])

    def drain_write(b):
        pltpu.make_async_copy(outs[b],
                              out_hbm.at[pl.ds(base, CHUNK)], ws[b]).wait()

    fire_gather(0, 0)
    fire_gather(1, 1)

    for b in range(2):
        wait_gather(b, b)
        _cast_chunk(rows[b], outs[b])
        fire_write(b, b)
        fire_gather(b + 2, b)

    half = n_chunk // 2

    def main_body(k, _):
        c = 2 * k
        for b in range(2):
            wait_gather(c + b, b)
            drain_write(b)
            _cast_chunk(rows[b], outs[b])
            fire_write(c + b, b)
            fire_gather(c + b + 2, b)
        return 0

    lax.fori_loop(1, half - 1, main_body, 0)

    for b in range(2):
        c = n_chunk - 2 + b
        wait_gather(c, b)
        drain_write(b)
        _cast_chunk(rows[b], outs[b])
        fire_write(c, b)

    drain_write(0)
    drain_write(1)


def _build(n_total, n_chunk):
    mesh = plsc.VectorSubcoreMesh(core_axis_name="c", subcore_axis_name="s")
    return pl.kernel(
        functools.partial(_emb_body, n_chunk),
        out_type=jax.ShapeDtypeStruct((n_total, D), jnp.bfloat16),
        mesh=mesh,
        scratch_types=[
            pltpu.VMEM((n_chunk, CHUNK), jnp.int32),
            pltpu.VMEM((CHUNK, D), jnp.float32),
            pltpu.VMEM((CHUNK, D), jnp.float32),
            pltpu.VMEM((CHUNK, D), jnp.bfloat16),
            pltpu.VMEM((CHUNK, D), jnp.bfloat16),
            pltpu.SemaphoreType.DMA,
            pltpu.SemaphoreType.DMA,
            pltpu.SemaphoreType.DMA,
            pltpu.SemaphoreType.DMA,
            pltpu.SemaphoreType.DMA,
        ],
        compiler_params=pltpu.CompilerParams(
            use_tc_tiling_on_sc=False, needs_layout_passes=False),
    )


def kernel(x, weight):
    orig_shape = x.shape
    n_total = x.size
    b_per_w = n_total // NW
    n_chunk = b_per_w // CHUNK
    xw = x.reshape(NW, n_chunk, CHUNK).astype(jnp.int32)
    out = _build(n_total, n_chunk)(xw, weight)
    return out.reshape(*orig_shape, D)


# speedup vs baseline: 1.1358x; 1.0734x over previous
import functools

import jax
import jax.numpy as jnp
from jax import lax
from jax.experimental import pallas as pl
from jax.experimental.pallas import tpu as pltpu
from jax.experimental.pallas import tpu_sc as plsc

D = 64
NC, NS = 2, 16
NW = NC * NS
CHUNK = 128
L = 16


def _cast_chunk(rows_ref, out_ref):
    iota = lax.iota(jnp.int32, L)
    even = 2 * iota
    odd = even + 1

    def row_body(r, _):
        rv = jnp.full((L,), r, jnp.int32)
        for h in range(D // (2 * L)):
            c0 = h * 2 * L
            e = plsc.load_gather(rows_ref, [rv, c0 + even])
            o = plsc.load_gather(rows_ref, [rv, c0 + odd])
            pk = plsc.pack(e, o, format=plsc.PackFormat.INTERLEAVED)
            out_ref[r, pl.ds(c0, 2 * L)] = pk
        return 0

    lax.fori_loop(0, CHUNK, row_body, 0, unroll=4)


def _emb_body(n_chunk, x_hbm, w_hbm, out_hbm, idx_v, r0, r1, o0, o1,
              sem_i, g0, g1, w0, w1):
    wid = lax.axis_index("s") * NC + lax.axis_index("c")
    base = wid * n_chunk * CHUNK
    pltpu.async_copy(x_hbm.at[wid], idx_v, sem_i).wait()

    rows = (r0, r1)
    outs = (o0, o1)
    gs = (g0, g1)
    ws = (w0, w1)

    def fire_gather(c, b):
        pltpu.async_copy(w_hbm.at[idx_v.at[c]], rows[b], gs[b])

    def wait_gather(c, b):
        pltpu.make_async_copy(w_hbm.at[idx_v.at[c]], rows[b], gs[b]).wait()

    def fire_write(c, b):
        pltpu.async_copy(outs[b], out_hbm.at[pl.ds(base + c * CHUNK, CHUNK)],
                         ws[b])

    def drain_write(b):
        pltpu.make_async_copy(outs[b],
                              out_hbm.at[pl.ds(base, CHUNK)], ws[b]).wait()

    fire_gather(0, 0)
    fire_gather(1, 1)

    for b in range(2):
        wait_gather(b, b)
        _cast_chunk(rows[b], outs[b])
        fire_write(b, b)
        fire_gather(b + 2, b)

    half = n_chunk // 2

    def main_body(k, _):
        c = 2 * k
        for b in range(2):
            wait_gather(c + b, b)
            drain_write(b)
            _cast_chunk(rows[b], outs[b])
            fire_write(c + b, b)
            fire_gather(c + b + 2, b)
        return 0

    lax.fori_loop(1, half - 1, main_body, 0)

    for b in range(2):
        c = n_chunk - 2 + b
        wait_gather(c, b)
        drain_write(b)
        _cast_chunk(rows[b], outs[b])
        fire_write(c, b)

    drain_write(0)
    drain_write(1)


def _build(n_total, n_chunk):
    mesh = plsc.VectorSubcoreMesh(core_axis_name="c", subcore_axis_name="s")
    return pl.kernel(
        functools.partial(_emb_body, n_chunk),
        out_type=jax.ShapeDtypeStruct((n_total, D), jnp.bfloat16),
        mesh=mesh,
        scratch_types=[
            pltpu.VMEM((n_chunk, CHUNK), jnp.int32),
            pltpu.VMEM((CHUNK, 128), jnp.float32),
            pltpu.VMEM((CHUNK, 128), jnp.float32),
            pltpu.VMEM((CHUNK, D), jnp.bfloat16),
            pltpu.VMEM((CHUNK, D), jnp.bfloat16),
            pltpu.SemaphoreType.DMA,
            pltpu.SemaphoreType.DMA,
            pltpu.SemaphoreType.DMA,
            pltpu.SemaphoreType.DMA,
            pltpu.SemaphoreType.DMA,
        ],
        compiler_params=pltpu.CompilerParams(
            use_tc_tiling_on_sc=False, needs_layout_passes=False),
    )


BN = 2048


def _tr_body(in_ref, out_ref):
    a = in_ref[...]
    out_ref[:, 0:D] = a.T


def _transpose_table(wt):
    n = wt.shape[1]
    return pl.pallas_call(
        _tr_body,
        grid=(pl.cdiv(n, BN),),
        in_specs=[pl.BlockSpec((D, BN), lambda i: (0, i))],
        out_specs=pl.BlockSpec((BN, 128), lambda i: (i, 0)),
        out_shape=jax.ShapeDtypeStruct((n, 128), jnp.float32),
    )(wt)


def kernel(x, weight):
    orig_shape = x.shape
    n_total = x.size
    b_per_w = n_total // NW
    n_chunk = b_per_w // CHUNK
    xw = x.reshape(NW, n_chunk, CHUNK).astype(jnp.int32)
    table = _transpose_table(weight.T)
    out = _build(n_total, n_chunk)(xw, table)
    return out.reshape(*orig_shape, D)
